# baseline (speedup 1.0000x reference)
import functools

import jax
import jax.numpy as jnp
from jax import lax
from jax.experimental import pallas as pl
from jax.experimental.pallas import tpu as pltpu, tpu_sc as plsc

_NC = 2
_NS = 16
_NW = _NC * _NS
_LANES = 16
_IDXW = 128


def _row_f32(ref, r, ev2):
    parts = []
    for v in range(ev2):
        packed = ref[r, pl.ds(v * 2 * _LANES, 2 * _LANES)]
        a, b = plsc.unpack(packed, format=plsc.PackFormat.INTERLEAVED,
                           preferred_element_type=jnp.float32)
        parts += [a, b]
    return parts


def _lane_perm(x, p):
    return lax.gather(
        x, p[:, None],
        lax.GatherDimensionNumbers(
            offset_dims=(), collapsed_slice_dims=(0,), start_index_map=(0,)),
        slice_sizes=(1,),
        mode=lax.GatherScatterMode.PROMISE_IN_BOUNDS)


def _build_sc_call(B, C, V, E):
    b_per_w = B // _NW
    CB = 128
    NCH = b_per_w // CB
    PB = CB * C
    TR = b_per_w // _IDXW
    CR = (b_per_w * C) // _IDXW
    CPC = CR // NCH
    EV2 = E // (2 * _LANES)

    mesh = plsc.VectorSubcoreMesh(core_axis_name="c", subcore_axis_name="s")

    @functools.partial(
        pl.kernel,
        out_type=jax.ShapeDtypeStruct((B * C,), jnp.float32),
        mesh=mesh,
        compiler_params=pltpu.CompilerParams(use_tc_tiling_on_sc=False,
                                             needs_layout_passes=False),
        scratch_types=[
            pltpu.VMEM((TR, _IDXW), jnp.int32),
            pltpu.VMEM((CR, _IDXW), jnp.int32),
            pltpu.VMEM((CB, E), jnp.bfloat16),
            pltpu.VMEM((PB, E), jnp.bfloat16),
            pltpu.VMEM((PB,), jnp.float32),
            pltpu.SemaphoreType.DMA,
        ],
    )
    def sc_call(tgt_hbm, ctx_hbm, ttab_hbm, ctab_hbm, out_hbm,
                tidx_v, cidx_v, trows_v, crows_v, outv, sem):
        wid = lax.axis_index("s") * _NC + lax.axis_index("c")
        staged = [
            pltpu.async_copy(
                tgt_hbm.at[pl.ds((wid * TR + j) * _IDXW, _IDXW)],
                tidx_v.at[j], sem)
            for j in range(TR)
        ] + [
            pltpu.async_copy(
                ctx_hbm.at[pl.ds((wid * CR + j) * _IDXW, _IDXW)],
                cidx_v.at[j], sem)
            for j in range(CR)
        ]
        for s in staged:
            s.wait()

        for k in range(NCH):
            started = [pltpu.async_copy(ttab_hbm.at[tidx_v.at[k]], trows_v, sem)]
            for j in range(CPC):
                started.append(pltpu.async_copy(
                    ctab_hbm.at[cidx_v.at[k * CPC + j]],
                    crows_v.at[pl.ds(j * _IDXW, _IDXW)], sem))
            for s in started:
                s.wait()

            lanes = lax.iota(jnp.int32, 16)
            perms = [lanes ^ sh for sh in (8, 4, 2, 1)]

            def gbody(g, carry):
                tb = g * _LANES
                pb = g * _LANES * C
                for j in range(C):
                    trow = {}
                    dotv = jnp.zeros((_LANES,), jnp.float32)
                    for i in range(_LANES):
                        q = j * _LANES + i
                        bi, _ = divmod(q, C)
                        if bi not in trow:
                            trow[bi] = _row_f32(trows_v, tb + bi, EV2)
                        tr = trow[bi]
                        cr = _row_f32(crows_v, pb + q, EV2)
                        acc = tr[0] * cr[0]
                        for v in range(1, 2 * EV2):
                            acc = acc + tr[v] * cr[v]
                        for pm in perms:
                            acc = acc + _lane_perm(acc, pm)
                        dotv = jnp.where(lanes == i, acc, dotv)
                    outv[pl.ds(pb + j * _LANES, _LANES)] = dotv
                return carry

            lax.fori_loop(0, CB // _LANES, gbody, 0)
            pltpu.sync_copy(
                outv, out_hbm.at[pl.ds((wid * NCH + k) * PB, PB)])

    return sc_call


def kernel(target, context, target_table, context_table):
    if target.ndim == 2:
        target = jnp.squeeze(target, axis=1)
    B = target.shape[0]
    C = context.shape[1]
    V, E = target_table.shape
    sc_call = _build_sc_call(B, C, V, E)
    tgt1 = target.astype(jnp.int32).reshape(B)
    ctx1 = context.astype(jnp.int32).reshape(B * C)
    ttab = target_table.astype(jnp.bfloat16)
    ctab = context_table.astype(jnp.bfloat16)
    out_flat = sc_call(tgt1, ctx1, ttab, ctab)
    return out_flat.reshape(B, C)


# speedup vs baseline: 1.2923x; 1.2923x over previous
import functools

import jax
import jax.numpy as jnp
from jax import lax
from jax.experimental import pallas as pl
from jax.experimental.pallas import tpu as pltpu, tpu_sc as plsc

_NC = 2
_NS = 16
_NW = _NC * _NS
_LANES = 16
_IDXW = 128


def _lane_perm(x, p):
    return lax.gather(
        x, p[:, None],
        lax.GatherDimensionNumbers(
            offset_dims=(), collapsed_slice_dims=(0,), start_index_map=(0,)),
        slice_sizes=(1,),
        mode=lax.GatherScatterMode.PROMISE_IN_BOUNDS)


def _build_sc_call(B, C, V, E):
    b_per_w = B // _NW
    CB = 128
    NCH = b_per_w // CB
    PB = CB * C
    NT = b_per_w
    NX = b_per_w * C
    EV = E // _LANES
    E2 = 2 * E

    mesh = plsc.VectorSubcoreMesh(core_axis_name="c", subcore_axis_name="s")

    @functools.partial(
        pl.kernel,
        out_type=jax.ShapeDtypeStruct((B * C,), jnp.float32),
        mesh=mesh,
        scratch_types=[
            pltpu.VMEM((NT,), jnp.int32),
            pltpu.VMEM((NX,), jnp.int32),
            pltpu.VMEM((NT,), jnp.int32),
            pltpu.VMEM((NX,), jnp.int32),
            pltpu.VMEM((CB, E2), jnp.float32),
            pltpu.VMEM((PB, E2), jnp.float32),
            pltpu.VMEM((PB,), jnp.float32),
            pltpu.SemaphoreType.DMA,
        ],
    )
    def sc_call(tgt_hbm, ctx_hbm, ttab_hbm, ctab_hbm, out_hbm,
                tidx_v, cidx_v, tsup_v, csup_v, trows_v, crows_v, outv, sem):
        wid = lax.axis_index("s") * _NC + lax.axis_index("c")
        staged = [
            pltpu.async_copy(tgt_hbm.at[pl.ds(wid * NT, NT)], tidx_v, sem),
            pltpu.async_copy(ctx_hbm.at[pl.ds(wid * NX, NX)], cidx_v, sem),
        ]
        for s in staged:
            s.wait()
        for l in range(NT // _LANES):
            tsup_v[pl.ds(l * _LANES, _LANES)] = (
                tidx_v[pl.ds(l * _LANES, _LANES)] >> 1)
        for l in range(NX // _LANES):
            csup_v[pl.ds(l * _LANES, _LANES)] = (
                cidx_v[pl.ds(l * _LANES, _LANES)] >> 1)

        for k in range(NCH):
            started = [pltpu.async_copy(
                ttab_hbm.at[tsup_v.at[pl.ds(k * CB, CB)]], trows_v, sem)]
            for j in range(PB // _IDXW):
                started.append(pltpu.async_copy(
                    ctab_hbm.at[csup_v.at[pl.ds(k * PB + j * _IDXW, _IDXW)]],
                    crows_v.at[pl.ds(j * _IDXW, _IDXW)], sem))
            for s in started:
                s.wait()

            lanes = lax.iota(jnp.int32, 16)
            perms = [lanes ^ sh for sh in (8, 4, 2, 1)]

            def gbody(g, carry):
                tb = g * _LANES
                pb = g * _LANES * C
                tparv = (tidx_v[pl.ds(k * CB + tb, _LANES)] & 1) * E
                for j in range(C):
                    trow = {}
                    dotv = jnp.zeros((_LANES,), jnp.float32)
                    cparv = (cidx_v[pl.ds(k * PB + pb + j * _LANES, _LANES)]
                             & 1) * E
                    for i in range(_LANES):
                        q = j * _LANES + i
                        bi, _ = divmod(q, C)
                        if bi not in trow:
                            toff = tparv[bi]
                            trow[bi] = [
                                trows_v[tb + bi, pl.ds(toff + v * _LANES, _LANES)]
                                for v in range(EV)]
                        tr = trow[bi]
                        coff = cparv[i]
                        acc = tr[0] * crows_v[pb + q, pl.ds(coff, _LANES)]
                        for v in range(1, EV):
                            acc = acc + tr[v] * crows_v[
                                pb + q, pl.ds(coff + v * _LANES, _LANES)]
                        for pm in perms:
                            acc = acc + _lane_perm(acc, pm)
                        dotv = jnp.where(lanes == i, acc, dotv)
                    outv[pl.ds(pb + j * _LANES, _LANES)] = dotv
                return carry

            lax.fori_loop(0, CB // _LANES, gbody, 0)
            pltpu.sync_copy(
                outv, out_hbm.at[pl.ds((wid * NCH + k) * PB, PB)])

    return sc_call


def kernel(target, context, target_table, context_table):
    if target.ndim == 2:
        target = jnp.squeeze(target, axis=1)
    B = target.shape[0]
    C = context.shape[1]
    V, E = target_table.shape
    sc_call = _build_sc_call(B, C, V, E)
    tgt1 = target.astype(jnp.int32).reshape(B)
    ctx1 = context.astype(jnp.int32).reshape(B * C)
    ttab2 = target_table.reshape(V // 2, 2 * E)
    ctab2 = context_table.reshape(V // 2, 2 * E)
    out_flat = sc_call(tgt1, ctx1, ttab2, ctab2)
    return out_flat.reshape(B, C)


# speedup vs baseline: 1.3758x; 1.0646x over previous
import functools

import jax
import jax.numpy as jnp
from jax import lax
from jax.experimental import pallas as pl
from jax.experimental.pallas import tpu as pltpu, tpu_sc as plsc

_NC = 2
_NS = 16
_NW = _NC * _NS
_LANES = 16
_IDXW = 128


def _lane_perm(x, p):
    return lax.gather(
        x, p[:, None],
        lax.GatherDimensionNumbers(
            offset_dims=(), collapsed_slice_dims=(0,), start_index_map=(0,)),
        slice_sizes=(1,),
        mode=lax.GatherScatterMode.PROMISE_IN_BOUNDS)


def _build_sc_call(B, C, V, E, EP):
    b_per_w = B // _NW
    CB = 128
    NCH = b_per_w // CB
    PB = CB * C
    NT = b_per_w
    NX = b_per_w * C
    EV = E // _LANES

    mesh = plsc.VectorSubcoreMesh(core_axis_name="c", subcore_axis_name="s")

    @functools.partial(
        pl.kernel,
        out_type=jax.ShapeDtypeStruct((B * C,), jnp.float32),
        mesh=mesh,
        scratch_types=[
            pltpu.VMEM((NT,), jnp.int32),
            pltpu.VMEM((NX,), jnp.int32),
            pltpu.VMEM((CB, EP), jnp.float32),
            pltpu.VMEM((PB, EP), jnp.float32),
            pltpu.VMEM((PB,), jnp.float32),
            pltpu.SemaphoreType.DMA,
        ],
    )
    def sc_call(tgt_hbm, ctx_hbm, ttab_hbm, ctab_hbm, out_hbm,
                tidx_v, cidx_v, trows_v, crows_v, outv, sem):
        wid = lax.axis_index("s") * _NC + lax.axis_index("c")
        staged = [
            pltpu.async_copy(tgt_hbm.at[pl.ds(wid * NT, NT)], tidx_v, sem),
            pltpu.async_copy(ctx_hbm.at[pl.ds(wid * NX, NX)], cidx_v, sem),
        ]
        for s in staged:
            s.wait()

        for k in range(NCH):
            started = [pltpu.async_copy(
                ttab_hbm.at[tidx_v.at[pl.ds(k * CB, CB)]], trows_v, sem)]
            for j in range(PB // _IDXW):
                started.append(pltpu.async_copy(
                    ctab_hbm.at[cidx_v.at[pl.ds(k * PB + j * _IDXW, _IDXW)]],
                    crows_v.at[pl.ds(j * _IDXW, _IDXW)], sem))
            for s in started:
                s.wait()

            lanes = lax.iota(jnp.int32, 16)
            perms = [lanes ^ sh for sh in (8, 4, 2, 1)]

            def gbody(g, carry):
                tb = g * _LANES
                pb = g * _LANES * C
                for j in range(C):
                    trow = {}
                    dotv = jnp.zeros((_LANES,), jnp.float32)
                    for i in range(_LANES):
                        q = j * _LANES + i
                        bi, _ = divmod(q, C)
                        if bi not in trow:
                            trow[bi] = [
                                trows_v[tb + bi, pl.ds(v * _LANES, _LANES)]
                                for v in range(EV)]
                        tr = trow[bi]
                        acc = tr[0] * crows_v[pb + q, pl.ds(0, _LANES)]
                        for v in range(1, EV):
                            acc = acc + tr[v] * crows_v[
                                pb + q, pl.ds(v * _LANES, _LANES)]
                        for pm in perms:
                            acc = acc + _lane_perm(acc, pm)
                        dotv = jnp.where(lanes == i, acc, dotv)
                    outv[pl.ds(pb + j * _LANES, _LANES)] = dotv
                return carry

            lax.fori_loop(0, CB // _LANES, gbody, 0)
            pltpu.sync_copy(
                outv, out_hbm.at[pl.ds((wid * NCH + k) * PB, PB)])

    return sc_call


def kernel(target, context, target_table, context_table):
    if target.ndim == 2:
        target = jnp.squeeze(target, axis=1)
    B = target.shape[0]
    C = context.shape[1]
    V, E = target_table.shape
    EP = 2 * E
    sc_call = _build_sc_call(B, C, V, E, EP)
    tgt1 = target.astype(jnp.int32).reshape(B)
    ctx1 = context.astype(jnp.int32).reshape(B * C)
    ttab2 = jnp.pad(target_table, ((0, 0), (0, EP - E)))
    ctab2 = jnp.pad(context_table, ((0, 0), (0, EP - E)))
    out_flat = sc_call(tgt1, ctx1, ttab2, ctab2)
    return out_flat.reshape(B, C)
